# baseline (speedup 1.0000x reference)
import functools

import jax
import jax.numpy as jnp
from jax import lax
from jax.experimental import pallas as pl
from jax.experimental.pallas import tpu as pltpu
from jax.experimental.pallas import tpu_sc as plsc

D_MODEL = 64
LANES = 16
GRP = 128
CHUNK_GROUPS = 4
CHUNK = GRP * CHUNK_GROUPS
NC = 2
NS = 16
NW = NC * NS


def _make_sc_kernel(T, n_groups):
    groups_per_worker = n_groups // NW
    n_chunks = groups_per_worker // CHUNK_GROUPS
    rows_per_worker = groups_per_worker * GRP
    pos_rows = T + CHUNK
    scale = jnp.float32(8.0)

    mesh = plsc.VectorSubcoreMesh(core_axis_name="c", subcore_axis_name="s")

    @functools.partial(
        pl.kernel,
        out_type=jax.ShapeDtypeStruct((n_groups * GRP, D_MODEL), jnp.float32),
        mesh=mesh,
        compiler_params=pltpu.CompilerParams(use_tc_tiling_on_sc=False),
        scratch_types=[
            pltpu.VMEM((CHUNK_GROUPS, GRP), jnp.int32),
            pltpu.VMEM((CHUNK, D_MODEL), jnp.float32),
            pltpu.VMEM((pos_rows, D_MODEL), jnp.float32),
            pltpu.SemaphoreType.DMA,
        ],
    )
    def sc_kernel(x_hbm, tok_hbm, pos_hbm, out_hbm, idx_v, rows_v, pos_v, gsem):
        wid = lax.axis_index("s") * NC + lax.axis_index("c")
        g_base = wid * groups_per_worker
        row_base = wid * rows_per_worker

        off = 0
        while off < pos_rows:
            n = min(T, pos_rows - off)
            pltpu.sync_copy(pos_hbm.at[pl.ds(0, n)], pos_v.at[pl.ds(off, n)])
            off += n

        def chunk_body(c, carry):
            g0 = g_base + c * CHUNK_GROUPS
            pltpu.sync_copy(x_hbm.at[pl.ds(g0, CHUNK_GROUPS)], idx_v)
            copies = []
            for j in range(CHUNK_GROUPS):
                copies.append(
                    pltpu.async_copy(
                        tok_hbm.at[idx_v.at[j]],
                        rows_v.at[pl.ds(j * GRP, GRP)],
                        gsem,
                    )
                )
            for cp in copies:
                cp.wait()

            pos_off = lax.rem(c * CHUNK, T)

            def row_body(r, carry2):
                pr = pos_off + r
                for j in range(D_MODEL // LANES):
                    s = pl.ds(j * LANES, LANES)
                    rows_v[r, s] = rows_v[r, s] * scale + pos_v[pr, s]
                return carry2

            lax.fori_loop(0, CHUNK, row_body, 0)
            pltpu.sync_copy(rows_v, out_hbm.at[pl.ds(row_base + c * CHUNK, CHUNK)])
            return carry

        lax.fori_loop(0, n_chunks, chunk_body, 0)

    return sc_kernel


def kernel(x, token_table, pos_table):
    B, T = x.shape
    N = B * T
    assert N % (NW * GRP) == 0
    assert (N // NW) % CHUNK == 0
    assert N // NW % T == 0
    x_flat = x.astype(jnp.int32).reshape(N // GRP, GRP)
    sc_kernel = _make_sc_kernel(T, N // GRP)
    out = sc_kernel(x_flat, token_table, pos_table)
    return out.reshape(B, T, D_MODEL)


# speedup vs baseline: 1.7110x; 1.7110x over previous
import functools

import jax
import jax.numpy as jnp
from jax import lax
from jax.experimental import pallas as pl
from jax.experimental.pallas import tpu as pltpu
from jax.experimental.pallas import tpu_sc as plsc

D_MODEL = 64
LANES = 16
GRP = 128
CHUNK_GROUPS = 2
CHUNK = GRP * CHUNK_GROUPS
NBUF = 4
NC = 2
NS = 16
NW = NC * NS


def _make_sc_kernel(T, n_groups):
    groups_per_worker = n_groups // NW
    n_chunks = groups_per_worker // CHUNK_GROUPS
    n4 = n_chunks // NBUF
    rows_per_worker = groups_per_worker * GRP
    pos_rows = T + CHUNK
    scale = jnp.float32(8.0)

    mesh = plsc.VectorSubcoreMesh(core_axis_name="c", subcore_axis_name="s")

    @functools.partial(
        pl.kernel,
        out_type=jax.ShapeDtypeStruct((n_groups * GRP, D_MODEL), jnp.float32),
        mesh=mesh,
        compiler_params=pltpu.CompilerParams(use_tc_tiling_on_sc=False),
        scratch_types=[
            pltpu.VMEM((NBUF, CHUNK_GROUPS, GRP), jnp.int32),
            pltpu.VMEM((NBUF, CHUNK, D_MODEL), jnp.float32),
            pltpu.VMEM((pos_rows, D_MODEL), jnp.float32),
        ] + [pltpu.SemaphoreType.DMA] * (3 * NBUF),
    )
    def sc_kernel(x_hbm, tok_hbm, pos_hbm, out_hbm, idx_v, rows_v, pos_v,
                  *sems):
        isem = sems[0:NBUF]
        gsem = sems[NBUF:2 * NBUF]
        osem = sems[2 * NBUF:3 * NBUF]
        wid = lax.axis_index("s") * NC + lax.axis_index("c")
        g_base = wid * groups_per_worker
        row_base = wid * rows_per_worker

        def idx_cp(c, b):
            return pltpu.make_async_copy(
                x_hbm.at[pl.ds(g_base + c * CHUNK_GROUPS, CHUNK_GROUPS)],
                idx_v.at[b], isem[b])

        def gather_cp(b, j):
            return pltpu.make_async_copy(
                tok_hbm.at[idx_v.at[b, j]],
                rows_v.at[b, pl.ds(j * GRP, GRP)], gsem[b])

        def out_cp(c, b):
            return pltpu.make_async_copy(
                rows_v.at[b],
                out_hbm.at[pl.ds(row_base + c * CHUNK, CHUNK)], osem[b])

        off = 0
        while off < pos_rows:
            n = min(T, pos_rows - off)
            pltpu.sync_copy(pos_hbm.at[pl.ds(0, n)], pos_v.at[pl.ds(off, n)])
            off += n

        for b in range(NBUF):
            idx_cp(b, b).start()
        idx_cp(0, 0).wait()
        for j in range(CHUNK_GROUPS):
            gather_cp(0, j).start()

        def ring_body(c4, carry):
            for b in range(NBUF):
                c = c4 * NBUF + b
                for j in range(CHUNK_GROUPS):
                    gather_cp(b, j).wait()

                @pl.when(c4 < n4 - 1)
                def _():
                    idx_cp(c + NBUF, b).start()

                b1 = (b + 1) % NBUF
                if b < NBUF - 1:
                    @pl.when(c - (NBUF - 1) >= 0)
                    def _():
                        out_cp(c - (NBUF - 1), b1).wait()
                    idx_cp(c + 1, b1).wait()
                    for j in range(CHUNK_GROUPS):
                        gather_cp(b1, j).start()
                else:
                    @pl.when(c4 < n4 - 1)
                    def _():
                        out_cp(c - (NBUF - 1), b1).wait()
                        idx_cp(c + 1, b1).wait()
                        for j in range(CHUNK_GROUPS):
                            gather_cp(b1, j).start()

                pos_off = lax.rem(c * CHUNK, T)

                @plsc.parallel_loop(0, CHUNK, unroll=2)
                def _(r):
                    pr = pos_off + r
                    for j in range(D_MODEL // LANES):
                        s = pl.ds(j * LANES, LANES)
                        rows_v[b, r, s] = rows_v[b, r, s] * scale + pos_v[pr, s]

                out_cp(c, b).start()
            return carry

        lax.fori_loop(0, n4, ring_body, 0)

        for k in range(NBUF):
            c = n_chunks - NBUF + k
            out_cp(c, c % NBUF).wait()

    return sc_kernel


def kernel(x, token_table, pos_table):
    B, T = x.shape
    N = B * T
    assert N % (NW * GRP) == 0
    assert (N // NW) % (CHUNK * NBUF) == 0
    assert N // NW % T == 0
    x_flat = x.astype(jnp.int32).reshape(N // GRP, GRP)
    sc_kernel = _make_sc_kernel(T, N // GRP)
    out = sc_kernel(x_flat, token_table, pos_table)
    return out.reshape(B, T, D_MODEL)


# speedup vs baseline: 1.7548x; 1.0256x over previous
import functools

import jax
import jax.numpy as jnp
from jax import lax
from jax.experimental import pallas as pl
from jax.experimental.pallas import tpu as pltpu
from jax.experimental.pallas import tpu_sc as plsc

D_MODEL = 64
LANES = 16
CHUNK_B = 2
NBUF = 4
NC = 2
NS = 16
NW = NC * NS


def _make_sc_kernel(B, T):
    b_per_worker = B // NW
    n_chunks = b_per_worker // CHUNK_B
    n4 = n_chunks // NBUF
    scale = jnp.float32(8.0)
    pieces = []
    off = 0
    while off < T:
        n = min(104, T - off)
        pieces.append((off, n))
        off += n
    groups = [(r2, off, n) for r2 in range(CHUNK_B) for (off, n) in pieces]

    mesh = plsc.VectorSubcoreMesh(core_axis_name="c", subcore_axis_name="s")

    @functools.partial(
        pl.kernel,
        out_type=jax.ShapeDtypeStruct((B, T, D_MODEL), jnp.float32),
        mesh=mesh,
        compiler_params=pltpu.CompilerParams(use_tc_tiling_on_sc=False),
        scratch_types=[
            pltpu.VMEM((NBUF, CHUNK_B, T), jnp.int32),
            pltpu.VMEM((NBUF, CHUNK_B, T, D_MODEL), jnp.float32),
            pltpu.VMEM((T, D_MODEL), jnp.float32),
        ] + [pltpu.SemaphoreType.DMA] * (3 * NBUF),
    )
    def sc_kernel(x_hbm, tok_hbm, pos_hbm, out_hbm, idx_v, rows_v, pos_v,
                  *sems):
        isem = sems[0:NBUF]
        gsem = sems[NBUF:2 * NBUF]
        osem = sems[2 * NBUF:3 * NBUF]
        wid = lax.axis_index("s") * NC + lax.axis_index("c")
        wb = wid * b_per_worker

        def idx_cp(c, b):
            return pltpu.make_async_copy(
                x_hbm.at[pl.ds(wb + c * CHUNK_B, CHUNK_B)],
                idx_v.at[b], isem[b])

        def gather_cp(b, g):
            r2, off, n = groups[g]
            return pltpu.make_async_copy(
                tok_hbm.at[idx_v.at[b, r2, pl.ds(off, n)]],
                rows_v.at[b, r2, pl.ds(off, n)], gsem[b])

        def out_cp(c, b):
            return pltpu.make_async_copy(
                rows_v.at[b],
                out_hbm.at[pl.ds(wb + c * CHUNK_B, CHUNK_B)], osem[b])

        pltpu.sync_copy(pos_hbm.at[pl.ds(0, T)], pos_v)

        for b in range(NBUF):
            idx_cp(b, b).start()
        idx_cp(0, 0).wait()
        for g in range(len(groups)):
            gather_cp(0, g).start()

        def ring_body(c4, carry):
            for b in range(NBUF):
                c = c4 * NBUF + b
                for g in range(len(groups)):
                    gather_cp(b, g).wait()

                @pl.when(c4 < n4 - 1)
                def _():
                    idx_cp(c + NBUF, b).start()

                b1 = (b + 1) % NBUF
                if b < NBUF - 1:
                    @pl.when(c - (NBUF - 1) >= 0)
                    def _():
                        out_cp(c - (NBUF - 1), b1).wait()
                    idx_cp(c + 1, b1).wait()
                    for g in range(len(groups)):
                        gather_cp(b1, g).start()
                else:
                    @pl.when(c4 < n4 - 1)
                    def _():
                        out_cp(c - (NBUF - 1), b1).wait()
                        idx_cp(c + 1, b1).wait()
                        for g in range(len(groups)):
                            gather_cp(b1, g).start()

                for r2 in range(CHUNK_B):
                    @plsc.parallel_loop(0, T, unroll=2)
                    def _(t):
                        for j in range(D_MODEL // LANES):
                            s = pl.ds(j * LANES, LANES)
                            rows_v[b, r2, t, s] = (
                                rows_v[b, r2, t, s] * scale + pos_v[t, s])

                out_cp(c, b).start()
            return carry

        lax.fori_loop(0, n4, ring_body, 0)

        for k in range(NBUF):
            c = n_chunks - NBUF + k
            out_cp(c, c % NBUF).wait()

    return sc_kernel


def kernel(x, token_table, pos_table):
    B, T = x.shape
    assert B % (NW * CHUNK_B * NBUF) == 0
    sc_kernel = _make_sc_kernel(B, T)
    return sc_kernel(x.astype(jnp.int32), token_table, pos_table)
